# baseline (speedup 1.0000x reference)
import functools

import jax
import jax.numpy as jnp
from jax import lax
from jax.experimental import pallas as pl
from jax.experimental.pallas import tpu as pltpu
from jax.experimental.pallas import tpu_sc as plsc

_NUM_CORES = 2
_NUM_SUBCORES = 16
_NUM_WORKERS = _NUM_CORES * _NUM_SUBCORES


@functools.partial(jax.jit, static_argnums=(2, 3, 4, 5))
def _gather(idx, table, B0, S, D, RB):
    rows_per_w = B0 // _NUM_WORKERS
    n_ch = rows_per_w // RB
    assert n_ch >= 2 and n_ch % 2 == 0 and n_ch * RB == rows_per_w
    mesh = plsc.VectorSubcoreMesh(core_axis_name="c", subcore_axis_name="s")

    @functools.partial(
        pl.kernel,
        mesh=mesh,
        out_type=jax.ShapeDtypeStruct((B0, S, D), jnp.float32),
        compiler_params=pltpu.CompilerParams(use_tc_tiling_on_sc=False),
        scratch_types=[
            pltpu.VMEM((rows_per_w, S), jnp.int32),
            pltpu.VMEM((RB, S, D), jnp.float32),
            pltpu.VMEM((RB, S, D), jnp.float32),
            pltpu.SemaphoreType.DMA,
            pltpu.SemaphoreType.DMA,
            pltpu.SemaphoreType.DMA,
            pltpu.SemaphoreType.DMA,
        ],
    )
    def k(idx_hbm, table_hbm, out_hbm, idx_v, rows0, rows1, g0, g1, s0, s1):
        wid = lax.axis_index("s") * _NUM_CORES + lax.axis_index("c")
        base = wid * rows_per_w
        pltpu.sync_copy(idx_hbm.at[pl.ds(base, rows_per_w)], idx_v)

        rows = (rows0, rows1)
        gsem = (g0, g1)
        ssem = (s0, s1)

        def start_g(i, b):
            for j in range(RB):
                pltpu.async_copy(
                    table_hbm.at[idx_v.at[i * RB + j]], rows[b].at[j], gsem[b]
                )

        def start_s(i, b):
            return pltpu.async_copy(
                rows[b], out_hbm.at[pl.ds(base + i * RB, RB)], ssem[b]
            )

        def wait_g(b):
            pltpu.make_async_copy(
                out_hbm.at[pl.ds(0, RB)], rows[b], gsem[b]
            ).wait()

        def wait_s(i, b):
            pltpu.make_async_copy(
                rows[b], out_hbm.at[pl.ds(base + i * RB, RB)], ssem[b]
            ).wait()

        start_g(0, 0)
        start_g(1, 1)
        wait_g(0)
        start_s(0, 0)

        T = n_ch // 2

        def round_body(t, carry):
            i0 = 2 * t
            wait_s(i0 - 2, 0)
            start_g(i0, 0)
            wait_g(1)
            start_s(i0 - 1, 1)
            wait_s(i0 - 1, 1)
            start_g(i0 + 1, 1)
            wait_g(0)
            start_s(i0, 0)
            return carry

        lax.fori_loop(1, T, round_body, 0)
        wait_g(1)
        start_s(n_ch - 1, 1)
        wait_s(n_ch - 2, 0)
        wait_s(n_ch - 1, 1)

    return k(idx, table)


def kernel(input_, weight):
    B0, S = input_.shape
    _, D = weight.shape
    return _gather(input_.astype(jnp.int32), weight, B0, S, D, 16)
